# Initial kernel scaffold; baseline (speedup 1.0000x reference)
import jax
import jax.numpy as jnp
from jax.experimental import pallas as pl


def kernel(x, gate_W, gate_b, exp_W1, exp_b1, exp_W2, exp_b2, W_out, b_out):
    raise NotImplementedError("write your pallas kernel here")


# speedup vs baseline: 1.0854x; 1.0854x over previous
import math

import jax
import jax.numpy as jnp
import numpy as np
from jax.experimental import pallas as pl
from jax.experimental.pallas import tpu as pltpu

B = 64
D_IN = 1024
H_PHASOR = 32
DELTA0 = 7.0
HIDDEN_DIM = 512
VOCAB_SIZE = 100000
NUM_EXPERTS = 8
EXPERT_DIM = 32
K_WINNERS = 5
DECAY = 0.7
THETA = 1.0
D_ENH = D_IN + 2 * H_PHASOR

VOCAB_TILE = 2048
NUM_TILES = (VOCAB_SIZE + VOCAB_TILE - 1) // VOCAB_TILE

def _gain_const(r):
    v = np.float32(1.0)
    for _ in range(r):
        v = np.float32(v * np.float32(DECAY))
    return float(1.0 + 1.0 / (1.0 + math.exp(-(float(v) - THETA))))

GAINS = [_gain_const(r) for r in range(K_WINNERS)]


def _body(x_ref, gw_ref, gb_ref, w1_ref, b1_ref, w2_ref, b2_ref,
          wout_ref, bout_ref, out_ref, att_ref):
    i = pl.program_id(0)

    @pl.when(i == 0)
    def _compute_attended():
        x = x_ref[...]
        xm = jnp.mean(x, axis=1, keepdims=True)
        h = (jax.lax.broadcasted_iota(jnp.int32, (1, H_PHASOR), 1)
             .astype(jnp.float32) + 1.0)
        phase = DELTA0 * xm * h
        xe = jnp.concatenate([x, jnp.cos(phase), jnp.sin(phase)], axis=1)

        logits = jnp.dot(xe, gw_ref[...],
                         preferred_element_type=jnp.float32) + gb_ref[...]
        iota_e = jax.lax.broadcasted_iota(jnp.int32, (B, NUM_EXPERTS), 1)
        m1 = jnp.max(logits, axis=1, keepdims=True)
        idx1 = jnp.min(jnp.where(logits == m1, iota_e, NUM_EXPERTS),
                       axis=1, keepdims=True)
        l2 = jnp.where(iota_e == idx1, -jnp.inf, logits)
        m2 = jnp.max(l2, axis=1, keepdims=True)
        idx2 = jnp.min(jnp.where(l2 == m2, iota_e, NUM_EXPERTS),
                       axis=1, keepdims=True)
        e2 = jnp.exp(m2 - m1)
        g1 = 1.0 / (1.0 + e2)
        g2 = e2 * g1
        wgt = (jnp.where(iota_e == idx1, g1, 0.0)
               + jnp.where(iota_e == idx2, g2, 0.0))

        ctx = jnp.zeros((B, HIDDEN_DIM), jnp.float32)
        for e in range(NUM_EXPERTS):
            he = jnp.maximum(
                jnp.dot(xe, w1_ref[e], preferred_element_type=jnp.float32)
                + b1_ref[e:e + 1, :], 0.0)
            oe = (jnp.dot(he, w2_ref[e], preferred_element_type=jnp.float32)
                  + b2_ref[e:e + 1, :])
            ctx = ctx + wgt[:, e:e + 1] * oe

        row = ctx[0:1, :]
        iota_v = jax.lax.broadcasted_iota(jnp.int32, (1, HIDDEN_DIM), 1)
        gvec = jnp.ones((1, HIDDEN_DIM), jnp.float32)
        for r in range(K_WINNERS):
            m = jnp.max(row, axis=1, keepdims=True)
            pos = jnp.min(jnp.where(row == m, iota_v, HIDDEN_DIM),
                          axis=1, keepdims=True)
            gvec = jnp.where(iota_v == pos, GAINS[r], gvec)
            row = jnp.where(iota_v == pos, -jnp.inf, row)

        att_ref[...] = ctx * gvec

    out_ref[...] = (jnp.dot(att_ref[...], wout_ref[...],
                            preferred_element_type=jnp.float32)
                    + bout_ref[...])


def kernel(x, gate_W, gate_b, exp_W1, exp_b1, exp_W2, exp_b2, W_out, b_out):
    gb2 = gate_b.reshape(1, NUM_EXPERTS)
    bout2 = b_out.reshape(1, VOCAB_SIZE)
    full = lambda shape: pl.BlockSpec(shape, lambda i: tuple(0 for _ in shape))
    return pl.pallas_call(
        _body,
        grid=(NUM_TILES,),
        in_specs=[
            full((B, D_IN)),
            full((D_ENH, NUM_EXPERTS)),
            full((1, NUM_EXPERTS)),
            full((NUM_EXPERTS, D_ENH, EXPERT_DIM)),
            full((NUM_EXPERTS, EXPERT_DIM)),
            full((NUM_EXPERTS, EXPERT_DIM, HIDDEN_DIM)),
            full((NUM_EXPERTS, HIDDEN_DIM)),
            pl.BlockSpec((HIDDEN_DIM, VOCAB_TILE), lambda i: (0, i)),
            pl.BlockSpec((1, VOCAB_TILE), lambda i: (0, i)),
        ],
        out_specs=pl.BlockSpec((B, VOCAB_TILE), lambda i: (0, i)),
        out_shape=jax.ShapeDtypeStruct((B, VOCAB_SIZE), jnp.float32),
        scratch_shapes=[pltpu.VMEM((B, HIDDEN_DIM), jnp.float32)],
        compiler_params=pltpu.CompilerParams(
            dimension_semantics=("arbitrary",)),
    )(x, gate_W, gb2, exp_W1, exp_b1, exp_W2, exp_b2, W_out, bout2)


# speedup vs baseline: 1.1174x; 1.0294x over previous
import math

import jax
import jax.numpy as jnp
import numpy as np
from jax.experimental import pallas as pl
from jax.experimental.pallas import tpu as pltpu

B = 64
D_IN = 1024
H_PHASOR = 32
DELTA0 = 7.0
HIDDEN_DIM = 512
VOCAB_SIZE = 100000
NUM_EXPERTS = 8
EXPERT_DIM = 32
K_WINNERS = 5
DECAY = 0.7
THETA = 1.0
D_ENH = D_IN + 2 * H_PHASOR

VOCAB_TILE = 4096
NUM_TILES = (VOCAB_SIZE + VOCAB_TILE - 1) // VOCAB_TILE

def _gain_const(r):
    v = np.float32(1.0)
    for _ in range(r):
        v = np.float32(v * np.float32(DECAY))
    return float(1.0 + 1.0 / (1.0 + math.exp(-(float(v) - THETA))))

GAINS = [_gain_const(r) for r in range(K_WINNERS)]


def _body(x_ref, gw_ref, gb_ref, w1_ref, b1_ref, w2_ref, b2_ref,
          wout_ref, bout_ref, out_ref, att_ref):
    i = pl.program_id(0)

    @pl.when(i == 0)
    def _compute_attended():
        x = x_ref[...]
        xm = jnp.mean(x, axis=1, keepdims=True)
        h = (jax.lax.broadcasted_iota(jnp.int32, (1, H_PHASOR), 1)
             .astype(jnp.float32) + 1.0)
        phase = DELTA0 * xm * h
        xe = jnp.concatenate([x, jnp.cos(phase), jnp.sin(phase)], axis=1)

        logits = jnp.dot(xe, gw_ref[...],
                         preferred_element_type=jnp.float32) + gb_ref[...]
        iota_e = jax.lax.broadcasted_iota(jnp.int32, (B, NUM_EXPERTS), 1)
        m1 = jnp.max(logits, axis=1, keepdims=True)
        idx1 = jnp.min(jnp.where(logits == m1, iota_e, NUM_EXPERTS),
                       axis=1, keepdims=True)
        l2 = jnp.where(iota_e == idx1, -jnp.inf, logits)
        m2 = jnp.max(l2, axis=1, keepdims=True)
        idx2 = jnp.min(jnp.where(l2 == m2, iota_e, NUM_EXPERTS),
                       axis=1, keepdims=True)
        e2 = jnp.exp(m2 - m1)
        g1 = 1.0 / (1.0 + e2)
        g2 = e2 * g1
        wgt = (jnp.where(iota_e == idx1, g1, 0.0)
               + jnp.where(iota_e == idx2, g2, 0.0))

        ctx = jnp.zeros((B, HIDDEN_DIM), jnp.float32)
        for e in range(NUM_EXPERTS):
            he = jnp.maximum(
                jnp.dot(xe, w1_ref[e], preferred_element_type=jnp.float32)
                + b1_ref[e:e + 1, :], 0.0)
            oe = (jnp.dot(he, w2_ref[e], preferred_element_type=jnp.float32)
                  + b2_ref[e:e + 1, :])
            ctx = ctx + wgt[:, e:e + 1] * oe

        row = ctx[0:1, :]
        iota_v = jax.lax.broadcasted_iota(jnp.int32, (1, HIDDEN_DIM), 1)
        gvec = jnp.ones((1, HIDDEN_DIM), jnp.float32)
        for r in range(K_WINNERS):
            m = jnp.max(row, axis=1, keepdims=True)
            pos = jnp.min(jnp.where(row == m, iota_v, HIDDEN_DIM),
                          axis=1, keepdims=True)
            gvec = jnp.where(iota_v == pos, GAINS[r], gvec)
            row = jnp.where(iota_v == pos, -jnp.inf, row)

        att_ref[...] = ctx * gvec

    out_ref[...] = (jnp.dot(att_ref[...], wout_ref[...],
                            preferred_element_type=jnp.float32)
                    + bout_ref[...])


def kernel(x, gate_W, gate_b, exp_W1, exp_b1, exp_W2, exp_b2, W_out, b_out):
    gb2 = gate_b.reshape(1, NUM_EXPERTS)
    bout2 = b_out.reshape(1, VOCAB_SIZE)
    full = lambda shape: pl.BlockSpec(shape, lambda i: tuple(0 for _ in shape))
    return pl.pallas_call(
        _body,
        grid=(NUM_TILES,),
        in_specs=[
            full((B, D_IN)),
            full((D_ENH, NUM_EXPERTS)),
            full((1, NUM_EXPERTS)),
            full((NUM_EXPERTS, D_ENH, EXPERT_DIM)),
            full((NUM_EXPERTS, EXPERT_DIM)),
            full((NUM_EXPERTS, EXPERT_DIM, HIDDEN_DIM)),
            full((NUM_EXPERTS, HIDDEN_DIM)),
            pl.BlockSpec((HIDDEN_DIM, VOCAB_TILE), lambda i: (0, i)),
            pl.BlockSpec((1, VOCAB_TILE), lambda i: (0, i)),
        ],
        out_specs=pl.BlockSpec((B, VOCAB_TILE), lambda i: (0, i)),
        out_shape=jax.ShapeDtypeStruct((B, VOCAB_SIZE), jnp.float32),
        scratch_shapes=[pltpu.VMEM((B, HIDDEN_DIM), jnp.float32)],
        compiler_params=pltpu.CompilerParams(
            dimension_semantics=("arbitrary",)),
    )(x, gate_W, gb2, exp_W1, exp_b1, exp_W2, exp_b2, W_out, bout2)


# speedup vs baseline: 1.1248x; 1.0066x over previous
import math

import jax
import jax.numpy as jnp
import numpy as np
from jax.experimental import pallas as pl
from jax.experimental.pallas import tpu as pltpu

B = 64
D_IN = 1024
H_PHASOR = 32
DELTA0 = 7.0
HIDDEN_DIM = 512
VOCAB_SIZE = 100000
NUM_EXPERTS = 8
EXPERT_DIM = 32
K_WINNERS = 5
DECAY = 0.7
THETA = 1.0
D_ENH = D_IN + 2 * H_PHASOR

VOCAB_TILE = 8192
NUM_TILES = (VOCAB_SIZE + VOCAB_TILE - 1) // VOCAB_TILE

def _gain_const(r):
    v = np.float32(1.0)
    for _ in range(r):
        v = np.float32(v * np.float32(DECAY))
    return float(1.0 + 1.0 / (1.0 + math.exp(-(float(v) - THETA))))

GAINS = [_gain_const(r) for r in range(K_WINNERS)]


def _body(x_ref, gw_ref, gb_ref, w1_ref, b1_ref, w2_ref, b2_ref,
          wout_ref, bout_ref, out_ref, att_ref):
    i = pl.program_id(0)

    @pl.when(i == 0)
    def _compute_attended():
        x = x_ref[...]
        xm = jnp.mean(x, axis=1, keepdims=True)
        h = (jax.lax.broadcasted_iota(jnp.int32, (1, H_PHASOR), 1)
             .astype(jnp.float32) + 1.0)
        phase = DELTA0 * xm * h
        xe = jnp.concatenate([x, jnp.cos(phase), jnp.sin(phase)], axis=1)

        logits = jnp.dot(xe, gw_ref[...],
                         preferred_element_type=jnp.float32) + gb_ref[...]
        iota_e = jax.lax.broadcasted_iota(jnp.int32, (B, NUM_EXPERTS), 1)
        m1 = jnp.max(logits, axis=1, keepdims=True)
        idx1 = jnp.min(jnp.where(logits == m1, iota_e, NUM_EXPERTS),
                       axis=1, keepdims=True)
        l2 = jnp.where(iota_e == idx1, -jnp.inf, logits)
        m2 = jnp.max(l2, axis=1, keepdims=True)
        idx2 = jnp.min(jnp.where(l2 == m2, iota_e, NUM_EXPERTS),
                       axis=1, keepdims=True)
        e2 = jnp.exp(m2 - m1)
        g1 = 1.0 / (1.0 + e2)
        g2 = e2 * g1
        wgt = (jnp.where(iota_e == idx1, g1, 0.0)
               + jnp.where(iota_e == idx2, g2, 0.0))

        ctx = jnp.zeros((B, HIDDEN_DIM), jnp.float32)
        for e in range(NUM_EXPERTS):
            he = jnp.maximum(
                jnp.dot(xe, w1_ref[e], preferred_element_type=jnp.float32)
                + b1_ref[e:e + 1, :], 0.0)
            oe = (jnp.dot(he, w2_ref[e], preferred_element_type=jnp.float32)
                  + b2_ref[e:e + 1, :])
            ctx = ctx + wgt[:, e:e + 1] * oe

        row = ctx[0:1, :]
        iota_v = jax.lax.broadcasted_iota(jnp.int32, (1, HIDDEN_DIM), 1)
        gvec = jnp.ones((1, HIDDEN_DIM), jnp.float32)
        for r in range(K_WINNERS):
            m = jnp.max(row, axis=1, keepdims=True)
            pos = jnp.min(jnp.where(row == m, iota_v, HIDDEN_DIM),
                          axis=1, keepdims=True)
            gvec = jnp.where(iota_v == pos, GAINS[r], gvec)
            row = jnp.where(iota_v == pos, -jnp.inf, row)

        att_ref[...] = ctx * gvec

    out_ref[...] = (jnp.dot(att_ref[...], wout_ref[...],
                            preferred_element_type=jnp.float32)
                    + bout_ref[...])


def kernel(x, gate_W, gate_b, exp_W1, exp_b1, exp_W2, exp_b2, W_out, b_out):
    gb2 = gate_b.reshape(1, NUM_EXPERTS)
    bout2 = b_out.reshape(1, VOCAB_SIZE)
    full = lambda shape: pl.BlockSpec(shape, lambda i: tuple(0 for _ in shape))
    return pl.pallas_call(
        _body,
        grid=(NUM_TILES,),
        in_specs=[
            full((B, D_IN)),
            full((D_ENH, NUM_EXPERTS)),
            full((1, NUM_EXPERTS)),
            full((NUM_EXPERTS, D_ENH, EXPERT_DIM)),
            full((NUM_EXPERTS, EXPERT_DIM)),
            full((NUM_EXPERTS, EXPERT_DIM, HIDDEN_DIM)),
            full((NUM_EXPERTS, HIDDEN_DIM)),
            pl.BlockSpec((HIDDEN_DIM, VOCAB_TILE), lambda i: (0, i)),
            pl.BlockSpec((1, VOCAB_TILE), lambda i: (0, i)),
        ],
        out_specs=pl.BlockSpec((B, VOCAB_TILE), lambda i: (0, i)),
        out_shape=jax.ShapeDtypeStruct((B, VOCAB_SIZE), jnp.float32),
        scratch_shapes=[pltpu.VMEM((B, HIDDEN_DIM), jnp.float32)],
        compiler_params=pltpu.CompilerParams(
            dimension_semantics=("arbitrary",)),
    )(x, gate_W, gb2, exp_W1, exp_b1, exp_W2, exp_b2, W_out, bout2)
